# baseline (speedup 1.0000x reference)
import jax
import jax.numpy as jnp
from jax import lax
from jax.experimental import pallas as pl
from jax.experimental.pallas import tpu as pltpu
from jax.experimental.pallas import tpu_sc as plsc

B, C, NUM, NCAT, V, D = 4096, 4, 13, 26, 100000, 32
PAIRS = B * C
N = PAIRS * NCAT
NC, NS = 2, 16
NW = NC * NS
PW = PAIRS // NW


def _sc_gather_body(s_catT_hbm, tables_hbm, out_hbm, idx_all, oidx_v, buf_v,
                    gsem, ssem):
    wid = lax.axis_index("s") * NC + lax.axis_index("c")
    base = wid * PW

    pltpu.sync_copy(s_catT_hbm.at[:, pl.ds(base, PW)], idx_all)

    for v in range(PW // 16):
        oidx_v[pl.ds(v * 16, 16)] = (base + v * 16 + lax.iota(jnp.int32, 16)) * NCAT

    def per_field(f, _):
        idx_f = idx_all.at[f]
        pltpu.async_copy(tables_hbm.at[f].at[idx_f], buf_v, gsem).wait()
        pltpu.async_copy(buf_v, out_hbm.at[oidx_v], ssem).wait()
        for v in range(PW // 16):
            sl = pl.ds(v * 16, 16)
            oidx_v[sl] = oidx_v[sl] + 1
        return _

    lax.fori_loop(0, NCAT, per_field, None)


@jax.jit
def _sc_gather(s_catT, cat_tables):
    mesh = plsc.VectorSubcoreMesh(core_axis_name="c", subcore_axis_name="s")
    return pl.kernel(
        _sc_gather_body,
        out_type=jax.ShapeDtypeStruct((N, D), jnp.float32),
        mesh=mesh,
        scratch_types=[
            pltpu.VMEM((NCAT, PW), jnp.int32),
            pltpu.VMEM((PW,), jnp.int32),
            pltpu.VMEM((PW, D), jnp.float32),
            pltpu.SemaphoreType.DMA,
            pltpu.SemaphoreType.DMA,
        ],
        compiler_params=pltpu.CompilerParams(use_tc_tiling_on_sc=False),
    )(s_catT, cat_tables)


BB = 128


def _assemble_body(s_cont_ref, w_ref, nan_ref, cat_ref, out_ref):
    sc = s_cont_ref[...]
    ne = sc[..., None] * w_ref[...][None, None]
    nan_mask = jnp.isnan(ne).all(axis=-1, keepdims=True)
    enc_cont = jnp.where(nan_mask, nan_ref[...][None, None], ne)
    cat = cat_ref[...].reshape(BB, C, NCAT, D)
    out_ref[...] = jnp.concatenate([enc_cont, cat], axis=2)


@jax.jit
def _assemble(s_cont, num_W, nan_embs, cat_rows):
    cat3 = cat_rows.reshape(B, C * NCAT, D)
    return pl.pallas_call(
        _assemble_body,
        grid=(B // BB,),
        in_specs=[
            pl.BlockSpec((BB, C, NUM), lambda i: (i, 0, 0)),
            pl.BlockSpec((NUM, D), lambda i: (0, 0)),
            pl.BlockSpec((NUM, D), lambda i: (0, 0)),
            pl.BlockSpec((BB, C * NCAT, D), lambda i: (i, 0, 0)),
        ],
        out_specs=pl.BlockSpec((BB, C, 39, D), lambda i: (i, 0, 0, 0)),
        out_shape=jax.ShapeDtypeStruct((B, C, NUM + NCAT, D), jnp.float32),
    )(s_cont, num_W, nan_embs, cat3)


def kernel(s_cont, s_cat, cat_tables, num_W, nan_embs):
    s_catT = s_cat.reshape(PAIRS, NCAT).T
    cat_rows = _sc_gather(s_catT, cat_tables)
    return _assemble(s_cont, num_W, nan_embs, cat_rows)


# speedup vs baseline: 4.2763x; 4.2763x over previous
import jax
import jax.numpy as jnp
from jax import lax
from jax.experimental import pallas as pl
from jax.experimental.pallas import tpu as pltpu
from jax.experimental.pallas import tpu_sc as plsc

B, C, NUM, NCAT, V, D = 4096, 4, 13, 26, 100000, 32
NC, NS = 2, 16
NW = NC * NS
GU = 8


def _sc_gather_body(s_catT_hbm, tablesT_hbm, out_hbm, slice_v, idx_v, val_v,
                    sem):
    wid = lax.axis_index("s") * NC + lax.axis_index("c")

    def per_field(f, _):
        pltpu.sync_copy(tablesT_hbm.at[f, wid], slice_v)

        def per_c(c, __):
            pltpu.sync_copy(s_catT_hbm.at[f, c], idx_v)

            def per_group(g, ___):
                for u in range(GU):
                    sl = pl.ds((g * GU + u) * 16, 16)
                    val_v[sl] = plsc.load_gather(slice_v, [idx_v[sl]])
                return ___

            lax.fori_loop(0, B // (16 * GU), per_group, None)
            pltpu.sync_copy(val_v, out_hbm.at[c, f, wid])
            return __

        lax.fori_loop(0, C, per_c, None)
        return _

    lax.fori_loop(0, NCAT, per_field, None)


@jax.jit
def _sc_gather(s_catT, tablesT):
    mesh = plsc.VectorSubcoreMesh(core_axis_name="c", subcore_axis_name="s")
    return pl.kernel(
        _sc_gather_body,
        out_type=jax.ShapeDtypeStruct((C, NCAT, D, B), jnp.float32),
        mesh=mesh,
        scratch_types=[
            pltpu.VMEM((V,), jnp.float32),
            pltpu.VMEM((B,), jnp.int32),
            pltpu.VMEM((B,), jnp.float32),
            pltpu.SemaphoreType.DMA,
        ],
        compiler_params=pltpu.CompilerParams(needs_layout_passes=False),
    )(s_catT, tablesT)


BBLK = 512
KD = (NUM + NCAT) * D


def _assemble_body(s_contT_ref, w_ref, nan_ref, cat_ref, out_ref):
    w = w_ref[...]
    nan_e = nan_ref[...]
    for c in range(C):
        sc = s_contT_ref[:, c, :]
        ne = sc[:, None, :] * w[:, :, None]
        nan_mask = jnp.isnan(sc)[:, None, :]
        enc = jnp.where(nan_mask, nan_e[:, :, None], ne)
        out_ref[pl.ds(c * KD, NUM * D), :] = enc.reshape(NUM * D, BBLK)
        cat = cat_ref[c]
        out_ref[pl.ds(c * KD + NUM * D, NCAT * D), :] = cat.reshape(
            NCAT * D, BBLK)


@jax.jit
def _assemble(s_contT, num_W, nan_embs, m_cat):
    return pl.pallas_call(
        _assemble_body,
        grid=(B // BBLK,),
        in_specs=[
            pl.BlockSpec((NUM, C, BBLK), lambda i: (0, 0, i)),
            pl.BlockSpec((NUM, D), lambda i: (0, 0)),
            pl.BlockSpec((NUM, D), lambda i: (0, 0)),
            pl.BlockSpec((C, NCAT, D, BBLK), lambda i: (0, 0, 0, i)),
        ],
        out_specs=pl.BlockSpec((C * KD, BBLK), lambda i: (0, i)),
        out_shape=jax.ShapeDtypeStruct((C * KD, B), jnp.float32),
    )(s_contT, num_W, nan_embs, m_cat)


def kernel(s_cont, s_cat, cat_tables, num_W, nan_embs):
    tablesT = jnp.transpose(cat_tables, (0, 2, 1))
    s_catT = jnp.transpose(s_cat, (2, 1, 0))
    s_contT = jnp.transpose(s_cont, (2, 1, 0))
    m_cat = _sc_gather(s_catT, tablesT)
    m2 = _assemble(s_contT, num_W, nan_embs, m_cat)
    out = jnp.transpose(m2.reshape(C, NUM + NCAT, D, B), (3, 0, 1, 2))
    return out
